# baseline (speedup 1.0000x reference)
import functools

import jax
import jax.numpy as jnp
from jax.experimental import pallas as pl
from jax.experimental.pallas import tpu as pltpu

T = 2048
D = 1024
H = 16
DK = 128
NE = 64
NS = 8
TOPK = 8
HD = H * DK
RB = 256
NRB = T // RB

def _dot(a, b, hi=False):
    if not hi:
        a, b = a.astype(jnp.bfloat16), b.astype(jnp.bfloat16)
    prec = jax.lax.Precision.HIGHEST if hi else jax.lax.Precision.DEFAULT
    return jax.lax.dot_general(a, b, (((1,), (0,)), ((), ())),
                               preferred_element_type=jnp.float32,
                               precision=prec)


def _dot_t(a, b, hi=False):
    if not hi:
        a, b = a.astype(jnp.bfloat16), b.astype(jnp.bfloat16)
    prec = jax.lax.Precision.HIGHEST if hi else jax.lax.Precision.DEFAULT
    return jax.lax.dot_general(a, b, (((1,), (1,)), ((), ())),
                               preferred_element_type=jnp.float32,
                               precision=prec)


def _proj_kernel(x_ref, w_ref, b_ref, y_ref):
    y_ref[...] = _dot(x_ref[...], w_ref[...]) + b_ref[...]


def _proj(x2d, Wct, bc):
    return pl.pallas_call(
        _proj_kernel,
        grid=(NRB,),
        in_specs=[
            pl.BlockSpec((RB, D), lambda i: (i, 0)),
            pl.BlockSpec((D, HD + 3 * D), lambda i: (0, 0)),
            pl.BlockSpec((1, HD + 3 * D), lambda i: (0, 0)),
        ],
        out_specs=pl.BlockSpec((RB, HD + 3 * D), lambda i: (i, 0)),
        out_shape=jax.ShapeDtypeStruct((T, HD + 3 * D), jnp.float32),
    )(x2d, Wct, bc)


def _bnstats_kernel(q_ref, mean_ref, rstd_ref):
    q = q_ref[...]
    mean = jnp.mean(q, axis=0, keepdims=True)
    var = jnp.mean(q * q, axis=0, keepdims=True) - mean * mean
    mean_ref[...] = mean
    rstd_ref[...] = jax.lax.rsqrt(var + 1e-5)


def _bnstats(y):
    return pl.pallas_call(
        _bnstats_kernel,
        grid=(1,),
        in_specs=[pl.BlockSpec((T, HD), lambda i: (0, 0))],
        out_specs=[pl.BlockSpec((1, HD), lambda i: (0, 0)),
                   pl.BlockSpec((1, HD), lambda i: (0, 0))],
        out_shape=[jax.ShapeDtypeStruct((1, HD), jnp.float32),
                   jax.ShapeDtypeStruct((1, HD), jnp.float32)],
    )(y)


def _peer_kernel(q_ref, mean_ref, rstd_ref, g_ref, beta_ref, C_ref, ewt_ref,
                 ew_ref, wot_ref, bout_ref, out_ref, acc):
    qn = (q_ref[...] - mean_ref[...]) * rstd_ref[...] * g_ref[...] + beta_ref[...]
    acc[...] = jnp.zeros_like(acc)
    for h in range(H):
        qh = qn[:, h * DK:(h + 1) * DK]
        scores = _dot(qh, C_ref[...])
        sim = _dot(qh, ewt_ref[...], hi=True)
        se = scores[:, :, None]
        sf = scores[:, None, :]
        gt = (sf > se).astype(jnp.float32)
        ee = jax.lax.broadcasted_iota(jnp.int32, (RB, NE, NE), 1)
        ff = jax.lax.broadcasted_iota(jnp.int32, (RB, NE, NE), 2)
        eq = ((sf == se) & (ff < ee)).astype(jnp.float32)
        rank = jnp.sum(gt + eq, axis=2)
        mask = rank < float(TOPK)
        m = jnp.max(jnp.where(mask, sim, -jnp.inf), axis=1, keepdims=True)
        p = jnp.where(mask, jnp.exp(sim - m), 0.0)
        rw = p / jnp.sum(p, axis=1, keepdims=True)
        oh = _dot(rw, ew_ref[...], hi=True)
        acc[...] += _dot(oh, wot_ref[h * DK:(h + 1) * DK, :])
    out_ref[...] = acc[...] + bout_ref[...]


def _peer(y, mean, rstd, gamma, beta, C, ewt, ew, Wot, bout):
    return pl.pallas_call(
        _peer_kernel,
        grid=(NRB,),
        in_specs=[
            pl.BlockSpec((RB, HD), lambda i: (i, 0)),
            pl.BlockSpec((1, HD), lambda i: (0, 0)),
            pl.BlockSpec((1, HD), lambda i: (0, 0)),
            pl.BlockSpec((1, HD), lambda i: (0, 0)),
            pl.BlockSpec((1, HD), lambda i: (0, 0)),
            pl.BlockSpec((DK, NE), lambda i: (0, 0)),
            pl.BlockSpec((DK, NE), lambda i: (0, 0)),
            pl.BlockSpec((NE, DK), lambda i: (0, 0)),
            pl.BlockSpec((HD, D), lambda i: (0, 0)),
            pl.BlockSpec((1, D), lambda i: (0, 0)),
        ],
        out_specs=pl.BlockSpec((RB, D), lambda i: (i, 0)),
        out_shape=jax.ShapeDtypeStruct((T, D), jnp.float32),
        scratch_shapes=[pltpu.VMEM((RB, D), jnp.float32)],
    )(y, mean, rstd, gamma, beta, C, ewt, ew, Wot, bout)


def _attn_kernel(qa_ref, ka_ref, va_ref, amean_ref, ao_ref):
    hp = pl.program_id(1)
    acc = jnp.zeros((RB, T), jnp.float32)
    for j in range(2):
        q = qa_ref[:, j * 64:(j + 1) * 64]
        k = ka_ref[:, j * 64:(j + 1) * 64]
        v = va_ref[:, j * 64:(j + 1) * 64]
        s = _dot_t(q, k) * 0.125
        m = jnp.max(s, axis=1, keepdims=True)
        p = jnp.exp(s - m)
        p = p / jnp.sum(p, axis=1, keepdims=True)
        acc = acc + p
        ao_ref[:, j * 64:(j + 1) * 64] = _dot(p, v)

    @pl.when(hp == 0)
    def _():
        amean_ref[...] = acc * (1.0 / H)

    @pl.when(hp != 0)
    def _():
        amean_ref[...] += acc * (1.0 / H)


def _attn(y):
    return pl.pallas_call(
        _attn_kernel,
        grid=(NRB, H // 2),
        in_specs=[
            pl.BlockSpec((RB, 128), lambda i, h: (i, 16 + h)),
            pl.BlockSpec((T, 128), lambda i, h: (0, 24 + h)),
            pl.BlockSpec((T, 128), lambda i, h: (0, 32 + h)),
        ],
        out_specs=[
            pl.BlockSpec((RB, T), lambda i, h: (i, 0)),
            pl.BlockSpec((RB, 128), lambda i, h: (i, h)),
        ],
        out_shape=[jax.ShapeDtypeStruct((T, T), jnp.float32),
                   jax.ShapeDtypeStruct((T, D), jnp.float32)],
    )(y, y, y)


def _final_kernel(x_ref, po_ref, ao_ref, wo_ref, bo_ref, rms_ref, out_ref):
    aop = _dot(ao_ref[...], wo_ref[...]) + bo_ref[...]
    hid = x_ref[...] + po_ref[...] + aop
    ms = jnp.mean(hid * hid, axis=1, keepdims=True)
    out_ref[...] = hid * jax.lax.rsqrt(ms + 1e-6) * rms_ref[...]


def _final(x2d, peer_out, ao, Wot2, bo, rms_w):
    return pl.pallas_call(
        _final_kernel,
        grid=(NRB,),
        in_specs=[
            pl.BlockSpec((RB, D), lambda i: (i, 0)),
            pl.BlockSpec((RB, D), lambda i: (i, 0)),
            pl.BlockSpec((RB, D), lambda i: (i, 0)),
            pl.BlockSpec((D, D), lambda i: (0, 0)),
            pl.BlockSpec((1, D), lambda i: (0, 0)),
            pl.BlockSpec((1, D), lambda i: (0, 0)),
        ],
        out_specs=pl.BlockSpec((RB, D), lambda i: (i, 0)),
        out_shape=jax.ShapeDtypeStruct((T, D), jnp.float32),
    )(x2d, peer_out, ao, Wot2, bo, rms_w)


def kernel(x, W_q, b_q, W_k, b_k, bn_gamma, bn_beta, sub_keys, expert_weights,
           W_out, b_out, W_in, b_in, W_o, b_o, rms_w):
    x2d = x.reshape(T, D)
    Wct = jnp.concatenate([W_q, W_in], axis=0).T
    bc = jnp.concatenate([b_q, b_in])[None, :]
    C = jnp.concatenate([jnp.repeat(sub_keys[0].T, NS, axis=1),
                         jnp.tile(sub_keys[1].T, (1, NS))], axis=0)
    ewt = expert_weights.T
    Wot = W_out.T
    Wot2 = W_o.T

    y = _proj(x2d, Wct, bc)
    mean, rstd = _bnstats(y)
    peer_out = _peer(y, mean, rstd, bn_gamma[None, :], bn_beta[None, :],
                     C, ewt, expert_weights, Wot, b_out[None, :])
    amean, ao = _attn(y)
    out = _final(x2d, peer_out, ao, Wot2, b_o[None, :], rms_w[None, :])
    return out.reshape(1, T, D), amean.reshape(1, T, T)


# speedup vs baseline: 3.9072x; 3.9072x over previous
import functools

import jax
import jax.numpy as jnp
from jax.experimental import pallas as pl
from jax.experimental.pallas import tpu as pltpu

T = 2048
D = 1024
H = 16
DK = 128
NE = 64
NS = 8
TOPK = 8
HD = H * DK
RB = 256
NRB = T // RB

def _dot(a, b, hi=False):
    if not hi:
        a, b = a.astype(jnp.bfloat16), b.astype(jnp.bfloat16)
    prec = jax.lax.Precision.HIGHEST if hi else jax.lax.Precision.DEFAULT
    return jax.lax.dot_general(a, b, (((1,), (0,)), ((), ())),
                               preferred_element_type=jnp.float32,
                               precision=prec)


def _dot_t(a, b, hi=False):
    if not hi:
        a, b = a.astype(jnp.bfloat16), b.astype(jnp.bfloat16)
    prec = jax.lax.Precision.HIGHEST if hi else jax.lax.Precision.DEFAULT
    return jax.lax.dot_general(a, b, (((1,), (1,)), ((), ())),
                               preferred_element_type=jnp.float32,
                               precision=prec)


def _proj_kernel(x_ref, w_ref, b_ref, y_ref):
    y_ref[...] = _dot(x_ref[...], w_ref[...]) + b_ref[...]


def _proj(x2d, Wct, bc):
    return pl.pallas_call(
        _proj_kernel,
        grid=(NRB,),
        in_specs=[
            pl.BlockSpec((RB, D), lambda i: (i, 0)),
            pl.BlockSpec((D, HD + 3 * D), lambda i: (0, 0)),
            pl.BlockSpec((1, HD + 3 * D), lambda i: (0, 0)),
        ],
        out_specs=pl.BlockSpec((RB, HD + 3 * D), lambda i: (i, 0)),
        out_shape=jax.ShapeDtypeStruct((T, HD + 3 * D), jnp.float32),
    )(x2d, Wct, bc)


def _bnstats_kernel(q_ref, mean_ref, rstd_ref):
    q = q_ref[...]
    mean = jnp.mean(q, axis=0, keepdims=True)
    var = jnp.mean(q * q, axis=0, keepdims=True) - mean * mean
    mean_ref[...] = mean
    rstd_ref[...] = jax.lax.rsqrt(var + 1e-5)


def _bnstats(y):
    return pl.pallas_call(
        _bnstats_kernel,
        grid=(1,),
        in_specs=[pl.BlockSpec((T, HD), lambda i: (0, 0))],
        out_specs=[pl.BlockSpec((1, HD), lambda i: (0, 0)),
                   pl.BlockSpec((1, HD), lambda i: (0, 0))],
        out_shape=[jax.ShapeDtypeStruct((1, HD), jnp.float32),
                   jax.ShapeDtypeStruct((1, HD), jnp.float32)],
    )(y)


def _peer_kernel(q_ref, mean_ref, rstd_ref, g_ref, beta_ref, C_ref, ewt_ref,
                 ew_ref, wot_ref, bout_ref, out_ref, acc):
    qn = (q_ref[...] - mean_ref[...]) * rstd_ref[...] * g_ref[...] + beta_ref[...]
    acc[...] = jnp.zeros_like(acc)
    for h in range(H):
        qh = qn[:, h * DK:(h + 1) * DK]
        scores = _dot(qh, C_ref[...])
        sim = _dot(qh, ewt_ref[...], hi=True)
        iota = jax.lax.broadcasted_iota(jnp.int32, (RB, NE), 1)
        work = scores
        mask = jnp.zeros((RB, NE), jnp.bool_)
        for _ in range(TOPK):
            m = jnp.max(work, axis=1, keepdims=True)
            cand = jnp.where(work == m, iota, NE)
            first = jnp.min(cand, axis=1, keepdims=True)
            pick = iota == first
            mask = jnp.logical_or(mask, pick)
            work = jnp.where(pick, -jnp.inf, work)
        m = jnp.max(jnp.where(mask, sim, -jnp.inf), axis=1, keepdims=True)
        p = jnp.where(mask, jnp.exp(sim - m), 0.0)
        rw = p / jnp.sum(p, axis=1, keepdims=True)
        oh = _dot(rw, ew_ref[...], hi=True)
        acc[...] += _dot(oh, wot_ref[h * DK:(h + 1) * DK, :])
    out_ref[...] = acc[...] + bout_ref[...]


def _peer(y, mean, rstd, gamma, beta, C, ewt, ew, Wot, bout):
    return pl.pallas_call(
        _peer_kernel,
        grid=(NRB,),
        in_specs=[
            pl.BlockSpec((RB, HD), lambda i: (i, 0)),
            pl.BlockSpec((1, HD), lambda i: (0, 0)),
            pl.BlockSpec((1, HD), lambda i: (0, 0)),
            pl.BlockSpec((1, HD), lambda i: (0, 0)),
            pl.BlockSpec((1, HD), lambda i: (0, 0)),
            pl.BlockSpec((DK, NE), lambda i: (0, 0)),
            pl.BlockSpec((DK, NE), lambda i: (0, 0)),
            pl.BlockSpec((NE, DK), lambda i: (0, 0)),
            pl.BlockSpec((HD, D), lambda i: (0, 0)),
            pl.BlockSpec((1, D), lambda i: (0, 0)),
        ],
        out_specs=pl.BlockSpec((RB, D), lambda i: (i, 0)),
        out_shape=jax.ShapeDtypeStruct((T, D), jnp.float32),
        scratch_shapes=[pltpu.VMEM((RB, D), jnp.float32)],
    )(y, mean, rstd, gamma, beta, C, ewt, ew, Wot, bout)


def _attn_kernel(qa_ref, ka_ref, va_ref, amean_ref, ao_ref):
    hp = pl.program_id(1)
    acc = jnp.zeros((RB, T), jnp.float32)
    for j in range(2):
        q = qa_ref[:, j * 64:(j + 1) * 64]
        k = ka_ref[:, j * 64:(j + 1) * 64]
        v = va_ref[:, j * 64:(j + 1) * 64]
        s = _dot_t(q, k) * 0.125
        m = jnp.max(s, axis=1, keepdims=True)
        p = jnp.exp(s - m)
        p = p / jnp.sum(p, axis=1, keepdims=True)
        acc = acc + p
        ao_ref[:, j * 64:(j + 1) * 64] = _dot(p, v)

    @pl.when(hp == 0)
    def _():
        amean_ref[...] = acc * (1.0 / H)

    @pl.when(hp != 0)
    def _():
        amean_ref[...] += acc * (1.0 / H)


def _attn(y):
    return pl.pallas_call(
        _attn_kernel,
        grid=(NRB, H // 2),
        in_specs=[
            pl.BlockSpec((RB, 128), lambda i, h: (i, 16 + h)),
            pl.BlockSpec((T, 128), lambda i, h: (0, 24 + h)),
            pl.BlockSpec((T, 128), lambda i, h: (0, 32 + h)),
        ],
        out_specs=[
            pl.BlockSpec((RB, T), lambda i, h: (i, 0)),
            pl.BlockSpec((RB, 128), lambda i, h: (i, h)),
        ],
        out_shape=[jax.ShapeDtypeStruct((T, T), jnp.float32),
                   jax.ShapeDtypeStruct((T, D), jnp.float32)],
    )(y, y, y)


def _final_kernel(x_ref, po_ref, ao_ref, wo_ref, bo_ref, rms_ref, out_ref):
    aop = _dot(ao_ref[...], wo_ref[...]) + bo_ref[...]
    hid = x_ref[...] + po_ref[...] + aop
    ms = jnp.mean(hid * hid, axis=1, keepdims=True)
    out_ref[...] = hid * jax.lax.rsqrt(ms + 1e-6) * rms_ref[...]


def _final(x2d, peer_out, ao, Wot2, bo, rms_w):
    return pl.pallas_call(
        _final_kernel,
        grid=(NRB,),
        in_specs=[
            pl.BlockSpec((RB, D), lambda i: (i, 0)),
            pl.BlockSpec((RB, D), lambda i: (i, 0)),
            pl.BlockSpec((RB, D), lambda i: (i, 0)),
            pl.BlockSpec((D, D), lambda i: (0, 0)),
            pl.BlockSpec((1, D), lambda i: (0, 0)),
            pl.BlockSpec((1, D), lambda i: (0, 0)),
        ],
        out_specs=pl.BlockSpec((RB, D), lambda i: (i, 0)),
        out_shape=jax.ShapeDtypeStruct((T, D), jnp.float32),
    )(x2d, peer_out, ao, Wot2, bo, rms_w)


def kernel(x, W_q, b_q, W_k, b_k, bn_gamma, bn_beta, sub_keys, expert_weights,
           W_out, b_out, W_in, b_in, W_o, b_o, rms_w):
    x2d = x.reshape(T, D)
    Wct = jnp.concatenate([W_q, W_in], axis=0).T
    bc = jnp.concatenate([b_q, b_in])[None, :]
    C = jnp.concatenate([jnp.repeat(sub_keys[0].T, NS, axis=1),
                         jnp.tile(sub_keys[1].T, (1, NS))], axis=0)
    ewt = expert_weights.T
    Wot = W_out.T
    Wot2 = W_o.T

    y = _proj(x2d, Wct, bc)
    mean, rstd = _bnstats(y)
    peer_out = _peer(y, mean, rstd, bn_gamma[None, :], bn_beta[None, :],
                     C, ewt, expert_weights, Wot, b_out[None, :])
    amean, ao = _attn(y)
    out = _final(x2d, peer_out, ao, Wot2, b_o[None, :], rms_w[None, :])
    return out.reshape(1, T, D), amean.reshape(1, T, T)


# speedup vs baseline: 4.2822x; 1.0960x over previous
import jax
import jax.numpy as jnp
from jax.experimental import pallas as pl
from jax.experimental.pallas import tpu as pltpu

T = 2048
D = 1024
H = 16
DK = 128
NE = 64
NS = 8
TOPK = 8
HD = H * DK
RB = 256
NRB = T // RB


def _dot(a, b, hi=False):
    if not hi:
        a, b = a.astype(jnp.bfloat16), b.astype(jnp.bfloat16)
    prec = jax.lax.Precision.HIGHEST if hi else jax.lax.Precision.DEFAULT
    return jax.lax.dot_general(a, b, (((1,), (0,)), ((), ())),
                               preferred_element_type=jnp.float32,
                               precision=prec)


def _dot_t(a, b, hi=False):
    if not hi:
        a, b = a.astype(jnp.bfloat16), b.astype(jnp.bfloat16)
    prec = jax.lax.Precision.HIGHEST if hi else jax.lax.Precision.DEFAULT
    return jax.lax.dot_general(a, b, (((1,), (1,)), ((), ())),
                               preferred_element_type=jnp.float32,
                               precision=prec)


def _proj_kernel(x_ref, wq_ref, win_ref, b_ref, y_ref):
    x = x_ref[...]
    y_ref[:, :HD] = _dot_t(x, wq_ref[...]) + b_ref[:, :HD]
    y_ref[:, HD:] = _dot_t(x, win_ref[...]) + b_ref[:, HD:]


def _proj(x2d, W_q, W_in, bc):
    return pl.pallas_call(
        _proj_kernel,
        grid=(NRB,),
        in_specs=[
            pl.BlockSpec((RB, D), lambda i: (i, 0)),
            pl.BlockSpec((HD, D), lambda i: (0, 0)),
            pl.BlockSpec((3 * D, D), lambda i: (0, 0)),
            pl.BlockSpec((1, HD + 3 * D), lambda i: (0, 0)),
        ],
        out_specs=pl.BlockSpec((RB, HD + 3 * D), lambda i: (i, 0)),
        out_shape=jax.ShapeDtypeStruct((T, HD + 3 * D), jnp.float32),
    )(x2d, W_q, W_in, bc)


def _bnstats_kernel(q_ref, mean_ref, rstd_ref):
    q = q_ref[...]
    mean = jnp.mean(q, axis=0, keepdims=True)
    var = jnp.mean(q * q, axis=0, keepdims=True) - mean * mean
    mean_ref[...] = mean
    rstd_ref[...] = jax.lax.rsqrt(var + 1e-5)


def _bnstats(y):
    return pl.pallas_call(
        _bnstats_kernel,
        grid=(1,),
        in_specs=[pl.BlockSpec((T, HD), lambda i: (0, 0))],
        out_specs=[pl.BlockSpec((1, HD), lambda i: (0, 0)),
                   pl.BlockSpec((1, HD), lambda i: (0, 0))],
        out_shape=[jax.ShapeDtypeStruct((1, HD), jnp.float32),
                   jax.ShapeDtypeStruct((1, HD), jnp.float32)],
    )(y)


def _peer_kernel(q_ref, mean_ref, rstd_ref, g_ref, beta_ref, C_ref, ew_ref,
                 wout_ref, bout_ref, out_ref, acc):
    qn = (q_ref[...] - mean_ref[...]) * rstd_ref[...] * g_ref[...] + beta_ref[...]
    acc[...] = jnp.zeros_like(acc)
    for h in range(H):
        qh = qn[:, h * DK:(h + 1) * DK]
        scores = _dot(qh, C_ref[...])
        sim = _dot_t(qh, ew_ref[...], hi=True)
        iota = jax.lax.broadcasted_iota(jnp.int32, (RB, NE), 1)
        work = scores
        mask = jnp.zeros((RB, NE), jnp.bool_)
        for _ in range(TOPK):
            m = jnp.max(work, axis=1, keepdims=True)
            cand = jnp.where(work == m, iota, NE)
            first = jnp.min(cand, axis=1, keepdims=True)
            pick = iota == first
            mask = jnp.logical_or(mask, pick)
            work = jnp.where(pick, -jnp.inf, work)
        m = jnp.max(jnp.where(mask, sim, -jnp.inf), axis=1, keepdims=True)
        p = jnp.where(mask, jnp.exp(sim - m), 0.0)
        rw = p * (1.0 / jnp.sum(p, axis=1, keepdims=True))
        oh = _dot(rw, ew_ref[...], hi=True)
        acc[...] += _dot_t(oh, wout_ref[:, h * DK:(h + 1) * DK])
    out_ref[...] = acc[...] + bout_ref[...]


def _peer(y, mean, rstd, gamma, beta, C, ew, W_out, bout):
    return pl.pallas_call(
        _peer_kernel,
        grid=(NRB,),
        in_specs=[
            pl.BlockSpec((RB, HD), lambda i: (i, 0)),
            pl.BlockSpec((1, HD), lambda i: (0, 0)),
            pl.BlockSpec((1, HD), lambda i: (0, 0)),
            pl.BlockSpec((1, HD), lambda i: (0, 0)),
            pl.BlockSpec((1, HD), lambda i: (0, 0)),
            pl.BlockSpec((DK, NE), lambda i: (0, 0)),
            pl.BlockSpec((NE, DK), lambda i: (0, 0)),
            pl.BlockSpec((D, HD), lambda i: (0, 0)),
            pl.BlockSpec((1, D), lambda i: (0, 0)),
        ],
        out_specs=pl.BlockSpec((RB, D), lambda i: (i, 0)),
        out_shape=jax.ShapeDtypeStruct((T, D), jnp.float32),
        scratch_shapes=[pltpu.VMEM((RB, D), jnp.float32)],
    )(y, mean, rstd, gamma, beta, C, ew, W_out, bout)


def _attn_kernel(qa_ref, ka_ref, va_ref, amean_ref, ao_ref):
    hp = pl.program_id(1)
    acc = jnp.zeros((RB, T), jnp.float32)
    for j in range(2):
        q = qa_ref[:, j * 64:(j + 1) * 64]
        k = ka_ref[:, j * 64:(j + 1) * 64]
        v = va_ref[:, j * 64:(j + 1) * 64]
        s = _dot_t(q, k) * 0.125
        m = jnp.max(s, axis=1, keepdims=True)
        p = jnp.exp(s - m)
        p = p * (1.0 / jnp.sum(p, axis=1, keepdims=True))
        acc = acc + p
        ao_ref[:, j * 64:(j + 1) * 64] = _dot(p, v)

    @pl.when(hp == 0)
    def _():
        amean_ref[...] = acc * (1.0 / H)

    @pl.when(hp != 0)
    def _():
        amean_ref[...] += acc * (1.0 / H)


def _attn(y):
    return pl.pallas_call(
        _attn_kernel,
        grid=(NRB, H // 2),
        in_specs=[
            pl.BlockSpec((RB, 128), lambda i, h: (i, 16 + h)),
            pl.BlockSpec((T, 128), lambda i, h: (0, 24 + h)),
            pl.BlockSpec((T, 128), lambda i, h: (0, 32 + h)),
        ],
        out_specs=[
            pl.BlockSpec((RB, T), lambda i, h: (i, 0)),
            pl.BlockSpec((RB, 128), lambda i, h: (i, h)),
        ],
        out_shape=[jax.ShapeDtypeStruct((T, T), jnp.float32),
                   jax.ShapeDtypeStruct((T, D), jnp.float32)],
    )(y, y, y)


def _final_kernel(x_ref, po_ref, ao_ref, wo_ref, bo_ref, rms_ref, out_ref):
    aop = _dot_t(ao_ref[...], wo_ref[...]) + bo_ref[...]
    hid = x_ref[...] + po_ref[...] + aop
    ms = jnp.mean(hid * hid, axis=1, keepdims=True)
    out_ref[...] = hid * jax.lax.rsqrt(ms + 1e-6) * rms_ref[...]


def _final(x2d, peer_out, ao, W_o, bo, rms_w):
    return pl.pallas_call(
        _final_kernel,
        grid=(NRB,),
        in_specs=[
            pl.BlockSpec((RB, D), lambda i: (i, 0)),
            pl.BlockSpec((RB, D), lambda i: (i, 0)),
            pl.BlockSpec((RB, D), lambda i: (i, 0)),
            pl.BlockSpec((D, D), lambda i: (0, 0)),
            pl.BlockSpec((1, D), lambda i: (0, 0)),
            pl.BlockSpec((1, D), lambda i: (0, 0)),
        ],
        out_specs=pl.BlockSpec((RB, D), lambda i: (i, 0)),
        out_shape=jax.ShapeDtypeStruct((T, D), jnp.float32),
    )(x2d, peer_out, ao, W_o, bo, rms_w)


def kernel(x, W_q, b_q, W_k, b_k, bn_gamma, bn_beta, sub_keys, expert_weights,
           W_out, b_out, W_in, b_in, W_o, b_o, rms_w):
    x2d = x.reshape(T, D)
    bc = jnp.concatenate([b_q, b_in])[None, :]
    C = jnp.concatenate([jnp.repeat(sub_keys[0].T, NS, axis=1),
                         jnp.tile(sub_keys[1].T, (1, NS))], axis=0)

    y = _proj(x2d, W_q, W_in, bc)
    mean, rstd = _bnstats(y)
    peer_out = _peer(y, mean, rstd, bn_gamma[None, :], bn_beta[None, :],
                     C, expert_weights, W_out, b_out[None, :])
    amean, ao = _attn(y)
    out = _final(x2d, peer_out, ao, W_o, b_o[None, :], rms_w[None, :])
    return out.reshape(1, T, D), amean.reshape(1, T, T)


# speedup vs baseline: 6.7341x; 1.5726x over previous
import jax
import jax.numpy as jnp
from jax.experimental import pallas as pl
from jax.experimental.pallas import tpu as pltpu

T = 2048
D = 1024
H = 16
DK = 128
NE = 64
NS = 8
TOPK = 8
HD = H * DK
RB = 256
NRB = T // RB


def _dot(a, b, hi=False):
    if not hi:
        a, b = a.astype(jnp.bfloat16), b.astype(jnp.bfloat16)
    prec = jax.lax.Precision.HIGHEST if hi else jax.lax.Precision.DEFAULT
    return jax.lax.dot_general(a, b, (((1,), (0,)), ((), ())),
                               preferred_element_type=jnp.float32,
                               precision=prec)


def _dot_t(a, b, hi=False):
    if not hi:
        a, b = a.astype(jnp.bfloat16), b.astype(jnp.bfloat16)
    prec = jax.lax.Precision.HIGHEST if hi else jax.lax.Precision.DEFAULT
    return jax.lax.dot_general(a, b, (((1,), (1,)), ((), ())),
                               preferred_element_type=jnp.float32,
                               precision=prec)


def _proj_kernel(x_ref, wq_ref, win_ref, b_ref, y_ref):
    x = x_ref[...]
    y_ref[:, :HD] = _dot_t(x, wq_ref[...]) + b_ref[:, :HD]
    y_ref[:, HD:] = _dot_t(x, win_ref[...]) + b_ref[:, HD:]


def _proj(x2d, W_q, W_in, bc):
    return pl.pallas_call(
        _proj_kernel,
        grid=(NRB,),
        in_specs=[
            pl.BlockSpec((RB, D), lambda i: (i, 0)),
            pl.BlockSpec((HD, D), lambda i: (0, 0)),
            pl.BlockSpec((3 * D, D), lambda i: (0, 0)),
            pl.BlockSpec((1, HD + 3 * D), lambda i: (0, 0)),
        ],
        out_specs=pl.BlockSpec((RB, HD + 3 * D), lambda i: (i, 0)),
        out_shape=jax.ShapeDtypeStruct((T, HD + 3 * D), jnp.float32),
    )(x2d, W_q, W_in, bc)


def _bnstats_kernel(q_ref, mean_ref, rstd_ref):
    q = q_ref[...]
    mean = jnp.mean(q, axis=0, keepdims=True)
    var = jnp.mean(q * q, axis=0, keepdims=True) - mean * mean
    mean_ref[...] = mean
    rstd_ref[...] = jax.lax.rsqrt(var + 1e-5)


def _bnstats(y):
    return pl.pallas_call(
        _bnstats_kernel,
        grid=(1,),
        in_specs=[pl.BlockSpec((T, HD), lambda i: (0, 0))],
        out_specs=[pl.BlockSpec((1, HD), lambda i: (0, 0)),
                   pl.BlockSpec((1, HD), lambda i: (0, 0))],
        out_shape=[jax.ShapeDtypeStruct((1, HD), jnp.float32),
                   jax.ShapeDtypeStruct((1, HD), jnp.float32)],
    )(y)


def _peer_kernel(q_ref, mean_ref, rstd_ref, g_ref, beta_ref, Ct_ref, ew_ref,
                 wout_ref, bout_ref, out_ref, acc, sct):
    qn = (q_ref[...] - mean_ref[...]) * rstd_ref[...] * g_ref[...] + beta_ref[...]
    for h in range(H):
        qh = qn[:, h * DK:(h + 1) * DK]
        sct[h] = _dot_t(Ct_ref[...], qh)
    work = sct[...]
    iota = jax.lax.broadcasted_iota(jnp.int32, (H, NE, RB), 1).astype(jnp.float32)
    madd = jnp.full((H, NE, RB), -jnp.inf, jnp.float32)
    for _ in range(TOPK):
        m = jnp.max(work, axis=1, keepdims=True)
        cand = jnp.where(work == m, iota, float(NE))
        first = jnp.min(cand, axis=1, keepdims=True)
        pick = iota == first
        madd = jnp.where(pick, 0.0, madd)
        work = jnp.where(pick, -jnp.inf, work)
    sct[...] = madd
    acc[...] = jnp.zeros_like(acc)
    for h in range(H):
        qh = qn[:, h * DK:(h + 1) * DK]
        simt = _dot_t(ew_ref[...], qh, hi=True)
        sm = simt + sct[h]
        m = jnp.max(sm, axis=0, keepdims=True)
        p = jnp.exp(sm - m)
        rwt = p * (1.0 / jnp.sum(p, axis=0, keepdims=True))
        oh = jax.lax.dot_general(rwt, ew_ref[...], (((0,), (0,)), ((), ())),
                                 preferred_element_type=jnp.float32,
                                 precision=jax.lax.Precision.HIGHEST)
        acc[...] += _dot_t(oh, wout_ref[:, h * DK:(h + 1) * DK])
    out_ref[...] = acc[...] + bout_ref[...]


def _peer(y, mean, rstd, gamma, beta, Ct, ew, W_out, bout):
    return pl.pallas_call(
        _peer_kernel,
        grid=(NRB,),
        in_specs=[
            pl.BlockSpec((RB, HD), lambda i: (i, 0)),
            pl.BlockSpec((1, HD), lambda i: (0, 0)),
            pl.BlockSpec((1, HD), lambda i: (0, 0)),
            pl.BlockSpec((1, HD), lambda i: (0, 0)),
            pl.BlockSpec((1, HD), lambda i: (0, 0)),
            pl.BlockSpec((NE, DK), lambda i: (0, 0)),
            pl.BlockSpec((NE, DK), lambda i: (0, 0)),
            pl.BlockSpec((D, HD), lambda i: (0, 0)),
            pl.BlockSpec((1, D), lambda i: (0, 0)),
        ],
        out_specs=pl.BlockSpec((RB, D), lambda i: (i, 0)),
        out_shape=jax.ShapeDtypeStruct((T, D), jnp.float32),
        scratch_shapes=[pltpu.VMEM((RB, D), jnp.float32),
                        pltpu.VMEM((H, NE, RB), jnp.float32)],
    )(y, mean, rstd, gamma, beta, Ct, ew, W_out, bout)


def _attn_kernel(qa_ref, ka_ref, va_ref, amean_ref, ao_ref):
    hp = pl.program_id(1)
    acc = jnp.zeros((RB, T), jnp.float32)
    for j in range(2):
        q = qa_ref[:, j * 64:(j + 1) * 64]
        k = ka_ref[:, j * 64:(j + 1) * 64]
        v = va_ref[:, j * 64:(j + 1) * 64]
        s = _dot_t(q, k) * 0.125
        m = jnp.max(s, axis=1, keepdims=True)
        p = jnp.exp(s - m)
        p = p * (1.0 / jnp.sum(p, axis=1, keepdims=True))
        acc = acc + p
        ao_ref[:, j * 64:(j + 1) * 64] = _dot(p, v)

    @pl.when(hp == 0)
    def _():
        amean_ref[...] = acc * (1.0 / H)

    @pl.when(hp != 0)
    def _():
        amean_ref[...] += acc * (1.0 / H)


def _attn(y):
    return pl.pallas_call(
        _attn_kernel,
        grid=(NRB, H // 2),
        in_specs=[
            pl.BlockSpec((RB, 128), lambda i, h: (i, 16 + h)),
            pl.BlockSpec((T, 128), lambda i, h: (0, 24 + h)),
            pl.BlockSpec((T, 128), lambda i, h: (0, 32 + h)),
        ],
        out_specs=[
            pl.BlockSpec((RB, T), lambda i, h: (i, 0)),
            pl.BlockSpec((RB, 128), lambda i, h: (i, h)),
        ],
        out_shape=[jax.ShapeDtypeStruct((T, T), jnp.float32),
                   jax.ShapeDtypeStruct((T, D), jnp.float32)],
    )(y, y, y)


def _final_kernel(x_ref, po_ref, ao_ref, wo_ref, bo_ref, rms_ref, out_ref):
    aop = _dot_t(ao_ref[...], wo_ref[...]) + bo_ref[...]
    hid = x_ref[...] + po_ref[...] + aop
    ms = jnp.mean(hid * hid, axis=1, keepdims=True)
    out_ref[...] = hid * jax.lax.rsqrt(ms + 1e-6) * rms_ref[...]


def _final(x2d, peer_out, ao, W_o, bo, rms_w):
    return pl.pallas_call(
        _final_kernel,
        grid=(NRB,),
        in_specs=[
            pl.BlockSpec((RB, D), lambda i: (i, 0)),
            pl.BlockSpec((RB, D), lambda i: (i, 0)),
            pl.BlockSpec((RB, D), lambda i: (i, 0)),
            pl.BlockSpec((D, D), lambda i: (0, 0)),
            pl.BlockSpec((1, D), lambda i: (0, 0)),
            pl.BlockSpec((1, D), lambda i: (0, 0)),
        ],
        out_specs=pl.BlockSpec((RB, D), lambda i: (i, 0)),
        out_shape=jax.ShapeDtypeStruct((T, D), jnp.float32),
    )(x2d, peer_out, ao, W_o, bo, rms_w)


def kernel(x, W_q, b_q, W_k, b_k, bn_gamma, bn_beta, sub_keys, expert_weights,
           W_out, b_out, W_in, b_in, W_o, b_o, rms_w):
    x2d = x.reshape(T, D)
    bc = jnp.concatenate([b_q, b_in])[None, :]
    Ct = jnp.concatenate([jnp.repeat(sub_keys[0], NS, axis=0),
                          jnp.tile(sub_keys[1], (NS, 1))], axis=1)

    y = _proj(x2d, W_q, W_in, bc)
    mean, rstd = _bnstats(y)
    peer_out = _peer(y, mean, rstd, bn_gamma[None, :], bn_beta[None, :],
                     Ct, expert_weights, W_out, b_out[None, :])
    amean, ao = _attn(y)
    out = _final(x2d, peer_out, ao, W_o, b_o[None, :], rms_w[None, :])
    return out.reshape(1, T, D), amean.reshape(1, T, T)
